# Initial kernel scaffold; baseline (speedup 1.0000x reference)
import jax
import jax.numpy as jnp
from jax.experimental import pallas as pl


def kernel(x, edge_index, batch, node_to_subgraph, subgraph_to_graph, W1, b1, W2, b2, W3, b3, Wl1, bl1, Wl2, bl2):
    raise NotImplementedError("write your pallas kernel here")


# speedup vs baseline: 13.1030x; 13.1030x over previous
import jax
import jax.numpy as jnp
from jax import lax
from jax.experimental import pallas as pl
from jax.experimental.pallas import tpu as pltpu
from jax.experimental.pallas import tpu_sc as plsc

N = 10000
E = 320000
H = 128
NSUBG = 2000
NGRAPH = 64
C = 10
NC = 2
NSC = 16
NW = NC * NSC
NPAD = 10240
SPAD = 2048
EK = 128
ECHUNKS = E // EK
PK = 80
PCHUNKS = N // PK
RB = 1000


def _fill1d(buf, n, val):
    v = jnp.full((16,), val, jnp.float32)

    def body(i, carry):
        buf[pl.ds(i * 16, 16)] = v
        return carry

    lax.fori_loop(0, n // 16, body, 0)


def _fill2d(buf, rows, val):
    v = jnp.full((16,), val, jnp.float32)

    def body(i, carry):
        r = i // 8
        col = (i % 8) * 16
        buf[r, pl.ds(col, 16)] = v
        return carry

    lax.fori_loop(0, rows * 8, body, 0)


def _worker_id():
    return lax.axis_index("s") * NC + lax.axis_index("c")


def _chunks_for(w, total):
    base, rem = total // NW, total % NW
    return jnp.where(w < rem, base + 1, base).astype(jnp.int32)


def _hist_body(dst_hbm, n2s_hbm, degp_hbm, cntp_hbm,
               idx_v, pidx_v, ones_v, zbuf, deg_sh, cnt_sh):
    cid = lax.axis_index("c")
    sid = lax.axis_index("s")
    w = _worker_id()

    _fill1d(ones_v, EK, 1.0)
    _fill1d(zbuf, 640, 0.0)
    pltpu.sync_copy(zbuf, deg_sh.at[pl.ds(sid * 640, 640)])
    pltpu.sync_copy(zbuf.at[pl.ds(0, 128)], cnt_sh.at[pl.ds(sid * 128, 128)])
    plsc.subcore_barrier()

    def ebody(i, carry):
        off = pl.multiple_of((w + i * NW) * EK, EK)
        pltpu.sync_copy(dst_hbm.at[pl.ds(off, EK)], idx_v)
        pltpu.sync_copy(ones_v, deg_sh.at[idx_v], add=True)
        return carry

    lax.fori_loop(0, _chunks_for(w, ECHUNKS), ebody, 0)

    def pbody(i, carry):
        off = pl.multiple_of((w + i * NW) * PK, PK)
        pltpu.sync_copy(n2s_hbm.at[pl.ds(off, PK)], pidx_v)
        pltpu.sync_copy(ones_v.at[pl.ds(0, PK)], cnt_sh.at[pidx_v], add=True)
        return carry

    lax.fori_loop(0, _chunks_for(w, PCHUNKS), pbody, 0)
    plsc.subcore_barrier()

    doff = pl.multiple_of(cid * NPAD + sid * 640, 128)
    pltpu.sync_copy(deg_sh.at[pl.ds(sid * 640, 640)], degp_hbm.at[pl.ds(doff, 640)])
    coff = pl.multiple_of(cid * SPAD + sid * 128, 128)
    pltpu.sync_copy(cnt_sh.at[pl.ds(sid * 128, 128)], cntp_hbm.at[pl.ds(coff, 128)])


_hist = pl.kernel(
    _hist_body,
    out_type=(jax.ShapeDtypeStruct((NC * NPAD,), jnp.float32),
              jax.ShapeDtypeStruct((NC * SPAD,), jnp.float32)),
    mesh=plsc.VectorSubcoreMesh(core_axis_name="c", subcore_axis_name="s"),
    scratch_types=(pltpu.VMEM((EK,), jnp.int32),
                   pltpu.VMEM((PK,), jnp.int32),
                   pltpu.VMEM((EK,), jnp.float32),
                   pltpu.VMEM((640,), jnp.float32),
                   pltpu.VMEM_SHARED((NPAD,), jnp.float32),
                   pltpu.VMEM_SHARED((SPAD,), jnp.float32)),
)


def _conv_body(g_hbm, src_hbm, dst_hbm, p_hbm,
               sidx_v, didx_v, rows_v, zbuf, acc_sh):
    cid = lax.axis_index("c")
    sid = lax.axis_index("s")
    w = _worker_id()

    _fill2d(zbuf, 128, 0.0)
    for k in range(5):
        pltpu.sync_copy(zbuf, acc_sh.at[pl.ds(sid * 640 + k * 128, 128)])
    plsc.subcore_barrier()

    def ebody(i, carry):
        off = pl.multiple_of((w + i * NW) * EK, EK)
        pltpu.sync_copy(src_hbm.at[pl.ds(off, EK)], sidx_v)
        pltpu.sync_copy(dst_hbm.at[pl.ds(off, EK)], didx_v)
        pltpu.sync_copy(g_hbm.at[sidx_v], rows_v)
        pltpu.sync_copy(rows_v, acc_sh.at[didx_v], add=True)
        return carry

    lax.fori_loop(0, _chunks_for(w, ECHUNKS), ebody, 0)
    plsc.subcore_barrier()

    hoff = pl.multiple_of(cid * NPAD + sid * 640, 128)
    pltpu.sync_copy(acc_sh.at[pl.ds(sid * 640, 640)], p_hbm.at[pl.ds(hoff, 640)])


_conv = pl.kernel(
    _conv_body,
    out_type=jax.ShapeDtypeStruct((NC * NPAD, H), jnp.float32),
    mesh=plsc.VectorSubcoreMesh(core_axis_name="c", subcore_axis_name="s"),
    scratch_types=(pltpu.VMEM((EK,), jnp.int32),
                   pltpu.VMEM((EK,), jnp.int32),
                   pltpu.VMEM((EK, H), jnp.float32),
                   pltpu.VMEM((128, H), jnp.float32),
                   pltpu.VMEM_SHARED((NPAD, H), jnp.float32)),
)


def _pool_body(h1_hbm, h2_hbm, h3_hbm, n2s_hbm, s1_hbm, s2_hbm, s3_hbm,
               pidx_v, r1, r2, r3, zbuf, a1, a2, a3):
    cid = lax.axis_index("c")
    sid = lax.axis_index("s")
    w = _worker_id()

    _fill2d(zbuf, 128, 0.0)
    for a in (a1, a2, a3):
        pltpu.sync_copy(zbuf, a.at[pl.ds(sid * 128, 128)])
    plsc.subcore_barrier()

    def pbody(i, carry):
        off = pl.multiple_of((w + i * NW) * PK, PK)
        pltpu.sync_copy(n2s_hbm.at[pl.ds(off, PK)], pidx_v)
        for h_hbm, r, a in ((h1_hbm, r1, a1), (h2_hbm, r2, a2), (h3_hbm, r3, a3)):
            pltpu.sync_copy(h_hbm.at[pl.ds(off, PK)], r)
            pltpu.sync_copy(r, a.at[pidx_v], add=True)
        return carry

    lax.fori_loop(0, _chunks_for(w, PCHUNKS), pbody, 0)
    plsc.subcore_barrier()

    hoff = pl.multiple_of(cid * SPAD + sid * 128, 128)
    for a, s_hbm in ((a1, s1_hbm), (a2, s2_hbm), (a3, s3_hbm)):
        pltpu.sync_copy(a.at[pl.ds(sid * 128, 128)], s_hbm.at[pl.ds(hoff, 128)])


_pool = pl.kernel(
    _pool_body,
    out_type=(jax.ShapeDtypeStruct((NC * SPAD, H), jnp.float32),) * 3,
    mesh=plsc.VectorSubcoreMesh(core_axis_name="c", subcore_axis_name="s"),
    scratch_types=(pltpu.VMEM((PK,), jnp.int32),
                   pltpu.VMEM((PK, H), jnp.float32),
                   pltpu.VMEM((PK, H), jnp.float32),
                   pltpu.VMEM((PK, H), jnp.float32),
                   pltpu.VMEM((128, H), jnp.float32),
                   pltpu.VMEM_SHARED((SPAD, H), jnp.float32),
                   pltpu.VMEM_SHARED((SPAD, H), jnp.float32),
                   pltpu.VMEM_SHARED((SPAD, H), jnp.float32)),
)


def _mm1_body(x_ref, w_ref, d0_ref, d1_ref, g_ref, dinv_ref):
    dinv = lax.rsqrt(1.0 + d0_ref[...] + d1_ref[...])
    g_ref[...] = jnp.dot(x_ref[...], w_ref[...],
                         preferred_element_type=jnp.float32) * dinv
    dinv_ref[...] = dinv


_mm1 = pl.pallas_call(
    _mm1_body,
    grid=(N // RB,),
    in_specs=[pl.BlockSpec((RB, H), lambda i: (i, 0)),
              pl.BlockSpec((H, H), lambda i: (0, 0)),
              pl.BlockSpec((RB, 1), lambda i: (i, 0)),
              pl.BlockSpec((RB, 1), lambda i: (i, 0))],
    out_specs=[pl.BlockSpec((RB, H), lambda i: (i, 0)),
               pl.BlockSpec((RB, 1), lambda i: (i, 0))],
    out_shape=[jax.ShapeDtypeStruct((N, H), jnp.float32),
               jax.ShapeDtypeStruct((N, 1), jnp.float32)],
)


def _comb_mm_body(p0_ref, p1_ref, g_ref, dinv_ref, b_ref, wn_ref, h_ref, gn_ref):
    dinv = dinv_ref[...]
    h = jnp.maximum(dinv * (p0_ref[...] + p1_ref[...] + g_ref[...]) + b_ref[...], 0.0)
    h_ref[...] = h
    gn_ref[...] = jnp.dot(h, wn_ref[...], preferred_element_type=jnp.float32) * dinv


_comb_mm = pl.pallas_call(
    _comb_mm_body,
    grid=(N // RB,),
    in_specs=[pl.BlockSpec((RB, H), lambda i: (i, 0)),
              pl.BlockSpec((RB, H), lambda i: (i, 0)),
              pl.BlockSpec((RB, H), lambda i: (i, 0)),
              pl.BlockSpec((RB, 1), lambda i: (i, 0)),
              pl.BlockSpec((1, H), lambda i: (0, 0)),
              pl.BlockSpec((H, H), lambda i: (0, 0))],
    out_specs=[pl.BlockSpec((RB, H), lambda i: (i, 0)),
               pl.BlockSpec((RB, H), lambda i: (i, 0))],
    out_shape=[jax.ShapeDtypeStruct((N, H), jnp.float32),
               jax.ShapeDtypeStruct((N, H), jnp.float32)],
)


def _comb_last_body(p0_ref, p1_ref, g_ref, dinv_ref, b_ref, h_ref):
    h_ref[...] = jnp.maximum(
        dinv_ref[...] * (p0_ref[...] + p1_ref[...] + g_ref[...]) + b_ref[...], 0.0)


_comb_last = pl.pallas_call(
    _comb_last_body,
    grid=(N // RB,),
    in_specs=[pl.BlockSpec((RB, H), lambda i: (i, 0)),
              pl.BlockSpec((RB, H), lambda i: (i, 0)),
              pl.BlockSpec((RB, H), lambda i: (i, 0)),
              pl.BlockSpec((RB, 1), lambda i: (i, 0)),
              pl.BlockSpec((1, H), lambda i: (0, 0))],
    out_specs=pl.BlockSpec((RB, H), lambda i: (i, 0)),
    out_shape=jax.ShapeDtypeStruct((N, H), jnp.float32),
)


def _head_body(s1_ref, s2_ref, s3_ref, c1_ref, s2g_ref,
               wl1_ref, bl1_ref, wl2_ref, bl2_ref, out_ref):
    xs = jnp.concatenate(
        [jnp.sum(s1_ref[...], axis=0)[:NSUBG],
         jnp.sum(s2_ref[...], axis=0)[:NSUBG],
         jnp.sum(s3_ref[...], axis=0)[:NSUBG]], axis=1)
    cnt1 = jnp.maximum(jnp.sum(c1_ref[...], axis=0)[:NSUBG], 1.0)
    sg = s2g_ref[...]
    sel = (sg == lax.broadcasted_iota(jnp.int32, (NSUBG, NGRAPH), 1))
    s_mat = jnp.where(sel, 1.0, 0.0).astype(jnp.float32)
    s_div = s_mat / cnt1
    xg = lax.dot_general(s_div, xs, (((0,), (0,)), ((), ())),
                         preferred_element_type=jnp.float32)
    cnt2 = lax.dot_general(s_mat, jnp.ones((NSUBG, 1), jnp.float32),
                           (((0,), (0,)), ((), ())),
                           preferred_element_type=jnp.float32)
    xg = xg / jnp.maximum(cnt2, 1.0)
    z = jnp.maximum(jnp.dot(xg, wl1_ref[...],
                            preferred_element_type=jnp.float32) + bl1_ref[...], 0.0)
    f = jnp.dot(z, wl2_ref[...],
                preferred_element_type=jnp.float32) + bl2_ref[...]
    m = jnp.max(f, axis=1, keepdims=True)
    out_ref[...] = f - (m + jnp.log(jnp.sum(jnp.exp(f - m), axis=1, keepdims=True)))


_head = pl.pallas_call(
    _head_body,
    out_shape=jax.ShapeDtypeStruct((NGRAPH, C), jnp.float32),
)


def kernel(x, edge_index, batch, node_to_subgraph, subgraph_to_graph,
           W1, b1, W2, b2, W3, b3, Wl1, bl1, Wl2, bl2):
    del batch
    src = edge_index[0]
    dst = edge_index[1]

    degp, cntp = _hist(dst, node_to_subgraph)
    d0 = degp[:N].reshape(N, 1)
    d1 = degp[NPAD:NPAD + N].reshape(N, 1)

    g1, dinv = _mm1(x, W1, d0, d1)

    p = _conv(g1, src, dst)
    h1, g2 = _comb_mm(p[:N], p[NPAD:NPAD + N], g1, dinv, b1.reshape(1, H), W2)

    p = _conv(g2, src, dst)
    h2, g3 = _comb_mm(p[:N], p[NPAD:NPAD + N], g2, dinv, b2.reshape(1, H), W3)

    p = _conv(g3, src, dst)
    h3 = _comb_last(p[:N], p[NPAD:NPAD + N], g3, dinv, b3.reshape(1, H))

    s1, s2, s3 = _pool(h1, h2, h3, node_to_subgraph)

    return _head(s1.reshape(NC, SPAD, H), s2.reshape(NC, SPAD, H),
                 s3.reshape(NC, SPAD, H), cntp.reshape(NC, SPAD, 1),
                 subgraph_to_graph.reshape(NSUBG, 1),
                 Wl1, bl1.reshape(1, H), Wl2, bl2.reshape(1, C))


# speedup vs baseline: 19.1280x; 1.4598x over previous
import jax
import jax.numpy as jnp
from jax import lax
from jax.experimental import pallas as pl
from jax.experimental.pallas import tpu as pltpu
from jax.experimental.pallas import tpu_sc as plsc

N = 10000
E = 320000
H = 128
NSUBG = 2000
NGRAPH = 64
C = 10
NC = 2
NSC = 16
NW = NC * NSC
NPAD = 10240
SPAD = 2048
EK = 128
ECHUNKS = E // EK
PK = 80
PCHUNKS = N // PK
RB = 1000


def _fill1d(buf, n, val):
    v = jnp.full((16,), val, jnp.float32)

    def body(i, carry):
        buf[pl.ds(i * 16, 16)] = v
        return carry

    lax.fori_loop(0, n // 16, body, 0)


def _fill2d(buf, rows, val):
    v = jnp.full((16,), val, jnp.float32)

    def body(i, carry):
        r = i // 8
        col = (i % 8) * 16
        buf[r, pl.ds(col, 16)] = v
        return carry

    lax.fori_loop(0, rows * 8, body, 0)


def _worker_id():
    return lax.axis_index("s") * NC + lax.axis_index("c")


def _chunks_for(w, total):
    base, rem = total // NW, total % NW
    return jnp.where(w < rem, base + 1, base).astype(jnp.int32)


def _range_for(w, total):
    base, rem = total // NW, total % NW
    start = (w * base + jnp.minimum(w, rem)).astype(jnp.int32)
    count = jnp.where(w < rem, base + 1, base).astype(jnp.int32)
    return start, count


def _hist_body(dst_hbm, n2s_hbm, degp_hbm, cntp_hbm,
               idx_v, pidx_v, ones_v, zbuf, deg_sh, cnt_sh, sem_s):
    cid = lax.axis_index("c")
    sid = lax.axis_index("s")
    w = _worker_id()

    _fill1d(ones_v, EK, 1.0)
    _fill1d(zbuf, 640, 0.0)
    pltpu.sync_copy(zbuf, deg_sh.at[pl.ds(sid * 640, 640)])
    pltpu.sync_copy(zbuf.at[pl.ds(0, 128)], cnt_sh.at[pl.ds(sid * 128, 128)])
    plsc.subcore_barrier()

    gbase, ng = _range_for(w, ECHUNKS)

    def scat_wait():
        pltpu.make_async_copy(ones_v, deg_sh.at[idx_v.at[0]], sem_s).wait()

    pltpu.sync_copy(dst_hbm.at[gbase], idx_v.at[0])

    def ebody(i, carry):
        b = i % 2

        @pl.when(i + 1 < ng)
        def _prefetch():
            @pl.when(i >= 1)
            def _():
                scat_wait()
            pltpu.sync_copy(dst_hbm.at[gbase + i + 1], idx_v.at[1 - b])

        pltpu.async_copy(ones_v, deg_sh.at[idx_v.at[b]], sem_s, add=True)
        return carry

    lax.fori_loop(0, ng, ebody, 0)
    scat_wait()
    scat_wait()

    def pbody(i, carry):
        off = pl.multiple_of((w + i * NW) * PK, PK)
        pltpu.sync_copy(n2s_hbm.at[pl.ds(off, PK)], pidx_v)
        pltpu.sync_copy(ones_v.at[pl.ds(0, PK)], cnt_sh.at[pidx_v], add=True)
        return carry

    lax.fori_loop(0, _chunks_for(w, PCHUNKS), pbody, 0)
    plsc.subcore_barrier()

    doff = pl.multiple_of(cid * NPAD + sid * 640, 128)
    pltpu.sync_copy(deg_sh.at[pl.ds(sid * 640, 640)], degp_hbm.at[pl.ds(doff, 640)])
    coff = pl.multiple_of(cid * SPAD + sid * 128, 128)
    pltpu.sync_copy(cnt_sh.at[pl.ds(sid * 128, 128)], cntp_hbm.at[pl.ds(coff, 128)])


_hist = pl.kernel(
    _hist_body,
    out_type=(jax.ShapeDtypeStruct((NC * NPAD,), jnp.float32),
              jax.ShapeDtypeStruct((NC * SPAD,), jnp.float32)),
    mesh=plsc.VectorSubcoreMesh(core_axis_name="c", subcore_axis_name="s"),
    scratch_types=(pltpu.VMEM((2, EK), jnp.int32),
                   pltpu.VMEM((PK,), jnp.int32),
                   pltpu.VMEM((EK,), jnp.float32),
                   pltpu.VMEM((640,), jnp.float32),
                   pltpu.VMEM_SHARED((NPAD,), jnp.float32),
                   pltpu.VMEM_SHARED((SPAD,), jnp.float32),
                   pltpu.SemaphoreType.DMA),
)


def _conv_body(g_hbm, src_hbm, dst_hbm, p_hbm,
               sidx, didx, rows, zbuf, acc_sh, sem_g, sem_s):
    cid = lax.axis_index("c")
    sid = lax.axis_index("s")
    w = _worker_id()

    _fill2d(zbuf, 64, 0.0)
    for k in range(10):
        pltpu.sync_copy(zbuf, acc_sh.at[pl.ds(sid * 640 + k * 64, 64)])
    plsc.subcore_barrier()

    gbase, ng = _range_for(w, ECHUNKS)

    def idx_load(slot, g):
        off = gbase + g
        pltpu.sync_copy(src_hbm.at[off], sidx.at[slot])
        pltpu.sync_copy(dst_hbm.at[off], didx.at[slot])

    def gather_wait():
        pltpu.make_async_copy(g_hbm.at[sidx.at[0]], rows.at[0], sem_g).wait()

    def scat_wait():
        pltpu.make_async_copy(rows.at[0], acc_sh.at[didx.at[0]], sem_s).wait()

    idx_load(0, 0)
    pltpu.async_copy(g_hbm.at[sidx.at[0]], rows.at[0], sem_g)

    def ebody(i, carry):
        b = i % 2

        @pl.when(i + 1 < ng)
        def _prefetch():
            @pl.when(i >= 1)
            def _():
                scat_wait()
            idx_load(1 - b, i + 1)
            pltpu.async_copy(g_hbm.at[sidx.at[1 - b]], rows.at[1 - b], sem_g)

        gather_wait()
        pltpu.async_copy(rows.at[b], acc_sh.at[didx.at[b]], sem_s, add=True)
        return carry

    lax.fori_loop(0, ng, ebody, 0)
    scat_wait()
    scat_wait()
    plsc.subcore_barrier()

    hoff = pl.multiple_of(cid * NPAD + sid * 640, 128)
    pltpu.sync_copy(acc_sh.at[pl.ds(sid * 640, 640)], p_hbm.at[pl.ds(hoff, 640)])


_conv = pl.kernel(
    _conv_body,
    out_type=jax.ShapeDtypeStruct((NC * NPAD, H), jnp.float32),
    mesh=plsc.VectorSubcoreMesh(core_axis_name="c", subcore_axis_name="s"),
    scratch_types=(pltpu.VMEM((2, EK), jnp.int32),
                   pltpu.VMEM((2, EK), jnp.int32),
                   pltpu.VMEM((2, EK, H), jnp.float32),
                   pltpu.VMEM((64, H), jnp.float32),
                   pltpu.VMEM_SHARED((NPAD, H), jnp.float32),
                   pltpu.SemaphoreType.DMA,
                   pltpu.SemaphoreType.DMA),
)


def _pool_body(h1_hbm, h2_hbm, h3_hbm, n2s_hbm, s1_hbm, s2_hbm, s3_hbm,
               pidx_v, r1, r2, r3, zbuf, a1, a2, a3):
    cid = lax.axis_index("c")
    sid = lax.axis_index("s")
    w = _worker_id()

    _fill2d(zbuf, 128, 0.0)
    for a in (a1, a2, a3):
        pltpu.sync_copy(zbuf, a.at[pl.ds(sid * 128, 128)])
    plsc.subcore_barrier()

    def pbody(i, carry):
        off = pl.multiple_of((w + i * NW) * PK, PK)
        pltpu.sync_copy(n2s_hbm.at[pl.ds(off, PK)], pidx_v)
        for h_hbm, r, a in ((h1_hbm, r1, a1), (h2_hbm, r2, a2), (h3_hbm, r3, a3)):
            pltpu.sync_copy(h_hbm.at[pl.ds(off, PK)], r)
            pltpu.sync_copy(r, a.at[pidx_v], add=True)
        return carry

    lax.fori_loop(0, _chunks_for(w, PCHUNKS), pbody, 0)
    plsc.subcore_barrier()

    hoff = pl.multiple_of(cid * SPAD + sid * 128, 128)
    for a, s_hbm in ((a1, s1_hbm), (a2, s2_hbm), (a3, s3_hbm)):
        pltpu.sync_copy(a.at[pl.ds(sid * 128, 128)], s_hbm.at[pl.ds(hoff, 128)])


_pool = pl.kernel(
    _pool_body,
    out_type=(jax.ShapeDtypeStruct((NC * SPAD, H), jnp.float32),) * 3,
    mesh=plsc.VectorSubcoreMesh(core_axis_name="c", subcore_axis_name="s"),
    scratch_types=(pltpu.VMEM((PK,), jnp.int32),
                   pltpu.VMEM((PK, H), jnp.float32),
                   pltpu.VMEM((PK, H), jnp.float32),
                   pltpu.VMEM((PK, H), jnp.float32),
                   pltpu.VMEM((128, H), jnp.float32),
                   pltpu.VMEM_SHARED((SPAD, H), jnp.float32),
                   pltpu.VMEM_SHARED((SPAD, H), jnp.float32),
                   pltpu.VMEM_SHARED((SPAD, H), jnp.float32)),
)


def _mm1_body(x_ref, w_ref, d0_ref, d1_ref, g_ref, dinv_ref):
    dinv = lax.rsqrt(1.0 + d0_ref[...] + d1_ref[...])
    g_ref[...] = jnp.dot(x_ref[...], w_ref[...],
                         preferred_element_type=jnp.float32) * dinv
    dinv_ref[...] = dinv


_mm1 = pl.pallas_call(
    _mm1_body,
    grid=(N // RB,),
    in_specs=[pl.BlockSpec((RB, H), lambda i: (i, 0)),
              pl.BlockSpec((H, H), lambda i: (0, 0)),
              pl.BlockSpec((RB, 1), lambda i: (i, 0)),
              pl.BlockSpec((RB, 1), lambda i: (i, 0))],
    out_specs=[pl.BlockSpec((RB, H), lambda i: (i, 0)),
               pl.BlockSpec((RB, 1), lambda i: (i, 0))],
    out_shape=[jax.ShapeDtypeStruct((N, H), jnp.float32),
               jax.ShapeDtypeStruct((N, 1), jnp.float32)],
)


def _comb_mm_body(p0_ref, p1_ref, g_ref, dinv_ref, b_ref, wn_ref, h_ref, gn_ref):
    dinv = dinv_ref[...]
    h = jnp.maximum(dinv * (p0_ref[...] + p1_ref[...] + g_ref[...]) + b_ref[...], 0.0)
    h_ref[...] = h
    gn_ref[...] = jnp.dot(h, wn_ref[...], preferred_element_type=jnp.float32) * dinv


_comb_mm = pl.pallas_call(
    _comb_mm_body,
    grid=(N // RB,),
    in_specs=[pl.BlockSpec((RB, H), lambda i: (i, 0)),
              pl.BlockSpec((RB, H), lambda i: (i, 0)),
              pl.BlockSpec((RB, H), lambda i: (i, 0)),
              pl.BlockSpec((RB, 1), lambda i: (i, 0)),
              pl.BlockSpec((1, H), lambda i: (0, 0)),
              pl.BlockSpec((H, H), lambda i: (0, 0))],
    out_specs=[pl.BlockSpec((RB, H), lambda i: (i, 0)),
               pl.BlockSpec((RB, H), lambda i: (i, 0))],
    out_shape=[jax.ShapeDtypeStruct((N, H), jnp.float32),
               jax.ShapeDtypeStruct((N, H), jnp.float32)],
)


def _comb_last_body(p0_ref, p1_ref, g_ref, dinv_ref, b_ref, h_ref):
    h_ref[...] = jnp.maximum(
        dinv_ref[...] * (p0_ref[...] + p1_ref[...] + g_ref[...]) + b_ref[...], 0.0)


_comb_last = pl.pallas_call(
    _comb_last_body,
    grid=(N // RB,),
    in_specs=[pl.BlockSpec((RB, H), lambda i: (i, 0)),
              pl.BlockSpec((RB, H), lambda i: (i, 0)),
              pl.BlockSpec((RB, H), lambda i: (i, 0)),
              pl.BlockSpec((RB, 1), lambda i: (i, 0)),
              pl.BlockSpec((1, H), lambda i: (0, 0))],
    out_specs=pl.BlockSpec((RB, H), lambda i: (i, 0)),
    out_shape=jax.ShapeDtypeStruct((N, H), jnp.float32),
)


def _head_body(s1_ref, s2_ref, s3_ref, c1_ref, s2g_ref,
               wl1_ref, bl1_ref, wl2_ref, bl2_ref, out_ref):
    xs = jnp.concatenate(
        [jnp.sum(s1_ref[...], axis=0)[:NSUBG],
         jnp.sum(s2_ref[...], axis=0)[:NSUBG],
         jnp.sum(s3_ref[...], axis=0)[:NSUBG]], axis=1)
    cnt1 = jnp.maximum(jnp.sum(c1_ref[...], axis=0)[:NSUBG], 1.0)
    sg = s2g_ref[...]
    sel = (sg == lax.broadcasted_iota(jnp.int32, (NSUBG, NGRAPH), 1))
    s_mat = jnp.where(sel, 1.0, 0.0).astype(jnp.float32)
    s_div = s_mat / cnt1
    xg = lax.dot_general(s_div, xs, (((0,), (0,)), ((), ())),
                         preferred_element_type=jnp.float32)
    cnt2 = lax.dot_general(s_mat, jnp.ones((NSUBG, 1), jnp.float32),
                           (((0,), (0,)), ((), ())),
                           preferred_element_type=jnp.float32)
    xg = xg / jnp.maximum(cnt2, 1.0)
    z = jnp.maximum(jnp.dot(xg, wl1_ref[...],
                            preferred_element_type=jnp.float32) + bl1_ref[...], 0.0)
    f = jnp.dot(z, wl2_ref[...],
                preferred_element_type=jnp.float32) + bl2_ref[...]
    m = jnp.max(f, axis=1, keepdims=True)
    out_ref[...] = f - (m + jnp.log(jnp.sum(jnp.exp(f - m), axis=1, keepdims=True)))


_head = pl.pallas_call(
    _head_body,
    out_shape=jax.ShapeDtypeStruct((NGRAPH, C), jnp.float32),
)


def kernel(x, edge_index, batch, node_to_subgraph, subgraph_to_graph,
           W1, b1, W2, b2, W3, b3, Wl1, bl1, Wl2, bl2):
    del batch
    src = edge_index[0].reshape(ECHUNKS, EK)
    dst = edge_index[1].reshape(ECHUNKS, EK)

    degp, cntp = _hist(dst, node_to_subgraph)
    d0 = degp[:N].reshape(N, 1)
    d1 = degp[NPAD:NPAD + N].reshape(N, 1)

    g1, dinv = _mm1(x, W1, d0, d1)

    p = _conv(g1, src, dst)
    h1, g2 = _comb_mm(p[:N], p[NPAD:NPAD + N], g1, dinv, b1.reshape(1, H), W2)

    p = _conv(g2, src, dst)
    h2, g3 = _comb_mm(p[:N], p[NPAD:NPAD + N], g2, dinv, b2.reshape(1, H), W3)

    p = _conv(g3, src, dst)
    h3 = _comb_last(p[:N], p[NPAD:NPAD + N], g3, dinv, b3.reshape(1, H))

    s1, s2, s3 = _pool(h1, h2, h3, node_to_subgraph)

    return _head(s1.reshape(NC, SPAD, H), s2.reshape(NC, SPAD, H),
                 s3.reshape(NC, SPAD, H), cntp.reshape(NC, SPAD, 1),
                 subgraph_to_graph.reshape(NSUBG, 1),
                 Wl1, bl1.reshape(1, H), Wl2, bl2.reshape(1, C))


# speedup vs baseline: 25.4461x; 1.3303x over previous
import jax
import jax.numpy as jnp
from jax import lax
from jax.experimental import pallas as pl
from jax.experimental.pallas import tpu as pltpu
from jax.experimental.pallas import tpu_sc as plsc

N = 10000
E = 320000
H = 128
NSUBG = 2000
NGRAPH = 64
C = 10
NC = 2
NSC = 16
NW = NC * NSC
NPAD = 10240
SPAD = 2048
EK = 128
ECHUNKS = E // EK
PK = 80
PCHUNKS = N // PK
RB = 1000


def _fill1d(buf, n, val):
    v = jnp.full((16,), val, jnp.float32)

    def body(i, carry):
        buf[pl.ds(i * 16, 16)] = v
        return carry

    lax.fori_loop(0, n // 16, body, 0)


def _fill2d(buf, rows, val):
    v = jnp.full((16,), val, jnp.float32)

    def body(i, carry):
        r = i // 8
        col = (i % 8) * 16
        buf[r, pl.ds(col, 16)] = v
        return carry

    lax.fori_loop(0, rows * 8, body, 0)


def _worker_id():
    return lax.axis_index("s") * NC + lax.axis_index("c")


def _chunks_for(w, total):
    base, rem = total // NW, total % NW
    return jnp.where(w < rem, base + 1, base).astype(jnp.int32)


def _range_for(w, total):
    base, rem = total // NW, total % NW
    start = (w * base + jnp.minimum(w, rem)).astype(jnp.int32)
    count = jnp.where(w < rem, base + 1, base).astype(jnp.int32)
    return start, count


def _hist_body(dst_hbm, n2s_hbm, degp_hbm, cntp_hbm,
               idx_v, pidx_v, ones_v, zbuf, deg_sh, cnt_sh, sem_i, sem_s):
    cid = lax.axis_index("c")
    sid = lax.axis_index("s")
    w = _worker_id()

    _fill1d(ones_v, EK, 1.0)
    _fill1d(zbuf, 640, 0.0)
    pltpu.sync_copy(zbuf, deg_sh.at[pl.ds(sid * 640, 640)])
    pltpu.sync_copy(zbuf.at[pl.ds(0, 128)], cnt_sh.at[pl.ds(sid * 128, 128)])
    plsc.subcore_barrier()

    gbase, ng = _range_for(w, ECHUNKS)

    def idx_start(g):
        pltpu.async_copy(dst_hbm.at[gbase + g], idx_v.at[g % 4], sem_i)

    def idx_wait():
        pltpu.make_async_copy(dst_hbm.at[gbase], idx_v.at[0], sem_i).wait()

    def scat_wait():
        pltpu.make_async_copy(ones_v, deg_sh.at[idx_v.at[0]], sem_s).wait()

    idx_start(0)
    idx_start(1)
    idx_start(2)

    def ebody(i, carry):
        @pl.when(i >= 1)
        def _():
            scat_wait()

        @pl.when(i + 3 < ng)
        def _():
            idx_start(i + 3)

        idx_wait()
        pltpu.async_copy(ones_v, deg_sh.at[idx_v.at[i % 4]], sem_s, add=True)
        return carry

    lax.fori_loop(0, ng, ebody, 0)
    scat_wait()

    def pbody(i, carry):
        off = pl.multiple_of((w + i * NW) * PK, PK)
        pltpu.sync_copy(n2s_hbm.at[pl.ds(off, PK)], pidx_v)
        pltpu.sync_copy(ones_v.at[pl.ds(0, PK)], cnt_sh.at[pidx_v], add=True)
        return carry

    lax.fori_loop(0, _chunks_for(w, PCHUNKS), pbody, 0)
    plsc.subcore_barrier()

    doff = pl.multiple_of(cid * NPAD + sid * 640, 128)
    pltpu.sync_copy(deg_sh.at[pl.ds(sid * 640, 640)], degp_hbm.at[pl.ds(doff, 640)])
    coff = pl.multiple_of(cid * SPAD + sid * 128, 128)
    pltpu.sync_copy(cnt_sh.at[pl.ds(sid * 128, 128)], cntp_hbm.at[pl.ds(coff, 128)])


_hist = pl.kernel(
    _hist_body,
    out_type=(jax.ShapeDtypeStruct((NC * NPAD,), jnp.float32),
              jax.ShapeDtypeStruct((NC * SPAD,), jnp.float32)),
    mesh=plsc.VectorSubcoreMesh(core_axis_name="c", subcore_axis_name="s"),
    scratch_types=(pltpu.VMEM((4, EK), jnp.int32),
                   pltpu.VMEM((PK,), jnp.int32),
                   pltpu.VMEM((EK,), jnp.float32),
                   pltpu.VMEM((640,), jnp.float32),
                   pltpu.VMEM_SHARED((NPAD,), jnp.float32),
                   pltpu.VMEM_SHARED((SPAD,), jnp.float32),
                   pltpu.SemaphoreType.DMA,
                   pltpu.SemaphoreType.DMA),
)


def _conv_body(g_hbm, src_hbm, dst_hbm, p_hbm,
               sidx, didx, rows, zbuf, acc_sh, sem_i, sem_g, sem_s):
    cid = lax.axis_index("c")
    sid = lax.axis_index("s")
    w = _worker_id()

    _fill2d(zbuf, 64, 0.0)
    for k in range(10):
        pltpu.sync_copy(zbuf, acc_sh.at[pl.ds(sid * 640 + k * 64, 64)])
    plsc.subcore_barrier()

    gbase, ng = _range_for(w, ECHUNKS)

    def idx_start(g):
        q = g % 4
        pltpu.async_copy(src_hbm.at[gbase + g], sidx.at[q], sem_i)
        pltpu.async_copy(dst_hbm.at[gbase + g], didx.at[q], sem_i)

    def idx_wait():
        pltpu.make_async_copy(src_hbm.at[gbase], sidx.at[0], sem_i).wait()
        pltpu.make_async_copy(dst_hbm.at[gbase], didx.at[0], sem_i).wait()

    def gather_start(g):
        pltpu.async_copy(g_hbm.at[sidx.at[g % 4]], rows.at[g % 2], sem_g)

    def gather_wait():
        pltpu.make_async_copy(g_hbm.at[sidx.at[0]], rows.at[0], sem_g).wait()

    def scat_wait():
        pltpu.make_async_copy(rows.at[0], acc_sh.at[didx.at[0]], sem_s).wait()

    idx_start(0)
    idx_start(1)
    idx_start(2)
    idx_wait()
    gather_start(0)

    def ebody(i, carry):
        b = i % 2

        @pl.when(i >= 1)
        def _():
            scat_wait()

        @pl.when(i + 3 < ng)
        def _():
            idx_start(i + 3)

        @pl.when(i + 1 < ng)
        def _():
            idx_wait()
            gather_start(i + 1)

        gather_wait()
        pltpu.async_copy(rows.at[b], acc_sh.at[didx.at[i % 4]], sem_s, add=True)
        return carry

    lax.fori_loop(0, ng, ebody, 0)
    scat_wait()
    plsc.subcore_barrier()

    hoff = pl.multiple_of(cid * NPAD + sid * 640, 128)
    pltpu.sync_copy(acc_sh.at[pl.ds(sid * 640, 640)], p_hbm.at[pl.ds(hoff, 640)])


_conv = pl.kernel(
    _conv_body,
    out_type=jax.ShapeDtypeStruct((NC * NPAD, H), jnp.float32),
    mesh=plsc.VectorSubcoreMesh(core_axis_name="c", subcore_axis_name="s"),
    scratch_types=(pltpu.VMEM((4, EK), jnp.int32),
                   pltpu.VMEM((4, EK), jnp.int32),
                   pltpu.VMEM((2, EK, H), jnp.float32),
                   pltpu.VMEM((64, H), jnp.float32),
                   pltpu.VMEM_SHARED((NPAD, H), jnp.float32),
                   pltpu.SemaphoreType.DMA,
                   pltpu.SemaphoreType.DMA,
                   pltpu.SemaphoreType.DMA),
)


def _pool_body(h1_hbm, h2_hbm, h3_hbm, n2s_hbm, s1_hbm, s2_hbm, s3_hbm,
               pidx_v, r1, r2, r3, zbuf, a1, a2, a3):
    cid = lax.axis_index("c")
    sid = lax.axis_index("s")
    w = _worker_id()

    _fill2d(zbuf, 128, 0.0)
    for a in (a1, a2, a3):
        pltpu.sync_copy(zbuf, a.at[pl.ds(sid * 128, 128)])
    plsc.subcore_barrier()

    def pbody(i, carry):
        off = pl.multiple_of((w + i * NW) * PK, PK)
        pltpu.sync_copy(n2s_hbm.at[pl.ds(off, PK)], pidx_v)
        for h_hbm, r, a in ((h1_hbm, r1, a1), (h2_hbm, r2, a2), (h3_hbm, r3, a3)):
            pltpu.sync_copy(h_hbm.at[pl.ds(off, PK)], r)
            pltpu.sync_copy(r, a.at[pidx_v], add=True)
        return carry

    lax.fori_loop(0, _chunks_for(w, PCHUNKS), pbody, 0)
    plsc.subcore_barrier()

    hoff = pl.multiple_of(cid * SPAD + sid * 128, 128)
    for a, s_hbm in ((a1, s1_hbm), (a2, s2_hbm), (a3, s3_hbm)):
        pltpu.sync_copy(a.at[pl.ds(sid * 128, 128)], s_hbm.at[pl.ds(hoff, 128)])


_pool = pl.kernel(
    _pool_body,
    out_type=(jax.ShapeDtypeStruct((NC * SPAD, H), jnp.float32),) * 3,
    mesh=plsc.VectorSubcoreMesh(core_axis_name="c", subcore_axis_name="s"),
    scratch_types=(pltpu.VMEM((PK,), jnp.int32),
                   pltpu.VMEM((PK, H), jnp.float32),
                   pltpu.VMEM((PK, H), jnp.float32),
                   pltpu.VMEM((PK, H), jnp.float32),
                   pltpu.VMEM((128, H), jnp.float32),
                   pltpu.VMEM_SHARED((SPAD, H), jnp.float32),
                   pltpu.VMEM_SHARED((SPAD, H), jnp.float32),
                   pltpu.VMEM_SHARED((SPAD, H), jnp.float32)),
)


def _mm1_body(x_ref, w_ref, d0_ref, d1_ref, g_ref, dinv_ref):
    dinv = lax.rsqrt(1.0 + d0_ref[...] + d1_ref[...])
    g_ref[...] = jnp.dot(x_ref[...], w_ref[...],
                         preferred_element_type=jnp.float32) * dinv
    dinv_ref[...] = dinv


_mm1 = pl.pallas_call(
    _mm1_body,
    grid=(N // RB,),
    in_specs=[pl.BlockSpec((RB, H), lambda i: (i, 0)),
              pl.BlockSpec((H, H), lambda i: (0, 0)),
              pl.BlockSpec((RB, 1), lambda i: (i, 0)),
              pl.BlockSpec((RB, 1), lambda i: (i, 0))],
    out_specs=[pl.BlockSpec((RB, H), lambda i: (i, 0)),
               pl.BlockSpec((RB, 1), lambda i: (i, 0))],
    out_shape=[jax.ShapeDtypeStruct((N, H), jnp.float32),
               jax.ShapeDtypeStruct((N, 1), jnp.float32)],
)


def _comb_mm_body(p0_ref, p1_ref, g_ref, dinv_ref, b_ref, wn_ref, h_ref, gn_ref):
    dinv = dinv_ref[...]
    h = jnp.maximum(dinv * (p0_ref[...] + p1_ref[...] + g_ref[...]) + b_ref[...], 0.0)
    h_ref[...] = h
    gn_ref[...] = jnp.dot(h, wn_ref[...], preferred_element_type=jnp.float32) * dinv


_comb_mm = pl.pallas_call(
    _comb_mm_body,
    grid=(N // RB,),
    in_specs=[pl.BlockSpec((RB, H), lambda i: (i, 0)),
              pl.BlockSpec((RB, H), lambda i: (i, 0)),
              pl.BlockSpec((RB, H), lambda i: (i, 0)),
              pl.BlockSpec((RB, 1), lambda i: (i, 0)),
              pl.BlockSpec((1, H), lambda i: (0, 0)),
              pl.BlockSpec((H, H), lambda i: (0, 0))],
    out_specs=[pl.BlockSpec((RB, H), lambda i: (i, 0)),
               pl.BlockSpec((RB, H), lambda i: (i, 0))],
    out_shape=[jax.ShapeDtypeStruct((N, H), jnp.float32),
               jax.ShapeDtypeStruct((N, H), jnp.float32)],
)


def _comb_last_body(p0_ref, p1_ref, g_ref, dinv_ref, b_ref, h_ref):
    h_ref[...] = jnp.maximum(
        dinv_ref[...] * (p0_ref[...] + p1_ref[...] + g_ref[...]) + b_ref[...], 0.0)


_comb_last = pl.pallas_call(
    _comb_last_body,
    grid=(N // RB,),
    in_specs=[pl.BlockSpec((RB, H), lambda i: (i, 0)),
              pl.BlockSpec((RB, H), lambda i: (i, 0)),
              pl.BlockSpec((RB, H), lambda i: (i, 0)),
              pl.BlockSpec((RB, 1), lambda i: (i, 0)),
              pl.BlockSpec((1, H), lambda i: (0, 0))],
    out_specs=pl.BlockSpec((RB, H), lambda i: (i, 0)),
    out_shape=jax.ShapeDtypeStruct((N, H), jnp.float32),
)


def _head_body(s1_ref, s2_ref, s3_ref, c1_ref, s2g_ref,
               wl1_ref, bl1_ref, wl2_ref, bl2_ref, out_ref):
    xs = jnp.concatenate(
        [jnp.sum(s1_ref[...], axis=0)[:NSUBG],
         jnp.sum(s2_ref[...], axis=0)[:NSUBG],
         jnp.sum(s3_ref[...], axis=0)[:NSUBG]], axis=1)
    cnt1 = jnp.maximum(jnp.sum(c1_ref[...], axis=0)[:NSUBG], 1.0)
    sg = s2g_ref[...]
    sel = (sg == lax.broadcasted_iota(jnp.int32, (NSUBG, NGRAPH), 1))
    s_mat = jnp.where(sel, 1.0, 0.0).astype(jnp.float32)
    s_div = s_mat / cnt1
    xg = lax.dot_general(s_div, xs, (((0,), (0,)), ((), ())),
                         preferred_element_type=jnp.float32)
    cnt2 = lax.dot_general(s_mat, jnp.ones((NSUBG, 1), jnp.float32),
                           (((0,), (0,)), ((), ())),
                           preferred_element_type=jnp.float32)
    xg = xg / jnp.maximum(cnt2, 1.0)
    z = jnp.maximum(jnp.dot(xg, wl1_ref[...],
                            preferred_element_type=jnp.float32) + bl1_ref[...], 0.0)
    f = jnp.dot(z, wl2_ref[...],
                preferred_element_type=jnp.float32) + bl2_ref[...]
    m = jnp.max(f, axis=1, keepdims=True)
    out_ref[...] = f - (m + jnp.log(jnp.sum(jnp.exp(f - m), axis=1, keepdims=True)))


_head = pl.pallas_call(
    _head_body,
    out_shape=jax.ShapeDtypeStruct((NGRAPH, C), jnp.float32),
)


def kernel(x, edge_index, batch, node_to_subgraph, subgraph_to_graph,
           W1, b1, W2, b2, W3, b3, Wl1, bl1, Wl2, bl2):
    del batch
    src = edge_index[0].reshape(ECHUNKS, EK)
    dst = edge_index[1].reshape(ECHUNKS, EK)

    degp, cntp = _hist(dst, node_to_subgraph)
    d0 = degp[:N].reshape(N, 1)
    d1 = degp[NPAD:NPAD + N].reshape(N, 1)

    g1, dinv = _mm1(x, W1, d0, d1)

    p = _conv(g1, src, dst)
    h1, g2 = _comb_mm(p[:N], p[NPAD:NPAD + N], g1, dinv, b1.reshape(1, H), W2)

    p = _conv(g2, src, dst)
    h2, g3 = _comb_mm(p[:N], p[NPAD:NPAD + N], g2, dinv, b2.reshape(1, H), W3)

    p = _conv(g3, src, dst)
    h3 = _comb_last(p[:N], p[NPAD:NPAD + N], g3, dinv, b3.reshape(1, H))

    s1, s2, s3 = _pool(h1, h2, h3, node_to_subgraph)

    return _head(s1.reshape(NC, SPAD, H), s2.reshape(NC, SPAD, H),
                 s3.reshape(NC, SPAD, H), cntp.reshape(NC, SPAD, 1),
                 subgraph_to_graph.reshape(NSUBG, 1),
                 Wl1, bl1.reshape(1, H), Wl2, bl2.reshape(1, C))
